# baseline (speedup 1.0000x reference)
import functools

import jax
import jax.numpy as jnp
from jax import lax
from jax.experimental import pallas as pl
from jax.experimental.pallas import tpu as pltpu
from jax.experimental.pallas import tpu_sc as plsc

_B, _L, _D = 16384, 200, 8
_NW = 32
_BS = _B // _NW
_LC = 8
_NCH = _L // _LC
_NBV = _BS // 16

_mesh = plsc.VectorSubcoreMesh(core_axis_name="c", subcore_axis_name="s")


@functools.partial(
    pl.kernel,
    mesh=_mesh,
    out_type=jax.ShapeDtypeStruct((_L * _D, _B), jnp.float32),
    scratch_types=[
        pltpu.VMEM((_D, 33), jnp.float32),
        pltpu.VMEM((2, _LC, _BS), jnp.int32),
        pltpu.VMEM((2, _LC * _D, _BS), jnp.float32),
        pltpu.SemaphoreType.DMA,
        pltpu.SemaphoreType.DMA,
    ],
    compiler_params=pltpu.CompilerParams(
        needs_layout_passes=False,
        disable_bounds_checks=True,
        skip_device_barrier=True,
    ),
)
def _lookup(xt_hbm, wt_hbm, out_hbm, wt_v, idx_v, stage_v, isem, osem):
    wid = lax.axis_index("s") * 2 + lax.axis_index("c")
    b0 = wid * _BS

    pltpu.sync_copy(wt_hbm, wt_v)

    dsplat = [jnp.full((16,), d, jnp.int32) for d in range(_D)]

    def issue_in(ci, buf):
        pltpu.async_copy(
            xt_hbm.at[pl.ds(ci * _LC, _LC), pl.ds(b0, _BS)],
            idx_v.at[buf],
            isem,
        )

    def wait_in(buf):
        pltpu.make_async_copy(
            xt_hbm.at[pl.ds(0, _LC), pl.ds(b0, _BS)], idx_v.at[buf], isem
        ).wait()

    def issue_out(ci, buf):
        pltpu.async_copy(
            stage_v.at[buf],
            out_hbm.at[pl.ds(ci * _LC * _D, _LC * _D), pl.ds(b0, _BS)],
            osem,
        )

    def wait_out(buf):
        pltpu.make_async_copy(
            stage_v.at[buf],
            out_hbm.at[pl.ds(0, _LC * _D), pl.ds(b0, _BS)],
            osem,
        ).wait()

    issue_in(0, 0)

    def chunk(ci, carry):
        buf = lax.rem(ci, 2)

        @pl.when(ci + 1 < _NCH)
        def _prefetch():
            issue_in(ci + 1, 1 - buf)

        wait_in(buf)

        @pl.when(ci >= 2)
        def _wait_prev_out():
            wait_out(buf)

        for l in range(_LC):
            @plsc.parallel_loop(0, _NBV, unroll=4)
            def _bv(k):
                bo = k * 16
                xv = idx_v[buf, l, pl.ds(bo, 16)]
                for d in range(_D):
                    val = plsc.load_gather(wt_v, [dsplat[d], xv])
                    stage_v[buf, l * _D + d, pl.ds(bo, 16)] = val

        issue_out(ci, buf)
        return carry

    lax.fori_loop(0, _NCH, chunk, 0)

    wait_out(lax.rem(_NCH - 2, 2))
    wait_out(lax.rem(_NCH - 1, 2))


def kernel(x, W):
    out2 = _lookup(x.T, W.T)
    return out2.reshape(_L, _D, _B).transpose(2, 0, 1)


# speedup vs baseline: 1.0004x; 1.0004x over previous
import functools

import jax
import jax.numpy as jnp
from jax import lax
from jax.experimental import pallas as pl
from jax.experimental.pallas import tpu as pltpu
from jax.experimental.pallas import tpu_sc as plsc

_B, _L, _D = 16384, 200, 8
_NW = 32
_BS = _B // _NW
_LC = 8
_NCH = _L // _LC
_NBV = _BS // 16

_mesh = plsc.VectorSubcoreMesh(core_axis_name="c", subcore_axis_name="s")


@functools.partial(
    pl.kernel,
    mesh=_mesh,
    out_type=jax.ShapeDtypeStruct((_L * _D, _B), jnp.float32),
    scratch_types=[
        pltpu.VMEM((_D, 33), jnp.float32),
        pltpu.VMEM((2, _LC, _BS), jnp.int32),
        pltpu.VMEM((2, _LC * _D, _BS), jnp.float32),
        pltpu.SemaphoreType.DMA,
        pltpu.SemaphoreType.DMA,
    ],
    compiler_params=pltpu.CompilerParams(
        needs_layout_passes=False,
        disable_bounds_checks=True,
    ),
)
def _lookup(xt_hbm, wt_hbm, out_hbm, wt_v, idx_v, stage_v, isem, osem):
    wid = lax.axis_index("s") * 2 + lax.axis_index("c")
    b0 = wid * _BS

    pltpu.sync_copy(wt_hbm, wt_v)

    dsplat = [jnp.full((16,), d, jnp.int32) for d in range(_D)]

    def issue_in(ci, buf):
        pltpu.async_copy(
            xt_hbm.at[pl.ds(ci * _LC, _LC), pl.ds(b0, _BS)],
            idx_v.at[buf],
            isem,
        )

    def wait_in(buf):
        pltpu.make_async_copy(
            xt_hbm.at[pl.ds(0, _LC), pl.ds(b0, _BS)], idx_v.at[buf], isem
        ).wait()

    def issue_out(ci, buf):
        pltpu.async_copy(
            stage_v.at[buf],
            out_hbm.at[pl.ds(ci * _LC * _D, _LC * _D), pl.ds(b0, _BS)],
            osem,
        )

    def wait_out(buf):
        pltpu.make_async_copy(
            stage_v.at[buf],
            out_hbm.at[pl.ds(0, _LC * _D), pl.ds(b0, _BS)],
            osem,
        ).wait()

    issue_in(0, 0)

    def chunk(ci, carry):
        buf = lax.rem(ci, 2)

        @pl.when(ci + 1 < _NCH)
        def _prefetch():
            issue_in(ci + 1, 1 - buf)

        wait_in(buf)

        @pl.when(ci >= 2)
        def _wait_prev_out():
            wait_out(buf)

        for l in range(_LC):
            @plsc.parallel_loop(0, _NBV, unroll=4)
            def _bv(k):
                bo = k * 16
                xv = idx_v[buf, l, pl.ds(bo, 16)]
                for d in range(_D):
                    val = plsc.load_gather(wt_v, [dsplat[d], xv])
                    stage_v[buf, l * _D + d, pl.ds(bo, 16)] = val

        issue_out(ci, buf)
        return carry

    lax.fori_loop(0, _NCH, chunk, 0)

    wait_out(lax.rem(_NCH - 2, 2))
    wait_out(lax.rem(_NCH - 1, 2))


def kernel(x, W):
    out2 = _lookup(x.T, W.T)
    return out2.reshape(_L, _D, _B).transpose(2, 0, 1)
